# baseline (speedup 1.0000x reference)
import functools

import jax
import jax.numpy as jnp
from jax import lax
from jax.experimental import pallas as pl
from jax.experimental.pallas import tpu as pltpu
from jax.experimental.pallas import tpu_sc as plsc

F = 26
V = 100000
K = 16
B = 4096
BF = B * F

NC = 2
NS = 16
NW = NC * NS
RPW = BF // NW
CHUNK = 128
NCHUNK = RPW // CHUNK


def _sc_gather_body(idx_hbm, emb_hbm, lin_hbm, emb_out, lin_out,
                    idx_v, rows_v, lin_v, sem):
    wid = lax.axis_index("s") * NC + lax.axis_index("c")
    base = pl.multiple_of(wid * RPW, RPW)
    pltpu.sync_copy(idx_hbm.at[pl.ds(base, RPW)], idx_v)

    def fire(c, carry):
        off = pl.multiple_of(c * CHUNK, CHUNK)
        idx_c = idx_v.at[pl.ds(off, CHUNK)]
        pltpu.async_copy(emb_hbm.at[idx_c], rows_v.at[pl.ds(off, CHUNK)], sem)
        pltpu.async_copy(lin_hbm.at[idx_c], lin_v.at[pl.ds(off, CHUNK)], sem)
        return carry

    lax.fori_loop(0, NCHUNK, fire, 0)

    def drain(c, carry):
        off = pl.multiple_of(c * CHUNK, CHUNK)
        idx_c = idx_v.at[pl.ds(off, CHUNK)]
        pltpu.make_async_copy(
            emb_hbm.at[idx_c], rows_v.at[pl.ds(off, CHUNK)], sem).wait()
        pltpu.make_async_copy(
            lin_hbm.at[idx_c], lin_v.at[pl.ds(off, CHUNK)], sem).wait()
        return carry

    lax.fori_loop(0, NCHUNK, drain, 0)

    pltpu.sync_copy(rows_v, emb_out.at[pl.ds(base, RPW)])
    pltpu.sync_copy(lin_v, lin_out.at[pl.ds(base, RPW)])


@functools.cache
def _sc_gather():
    return pl.kernel(
        _sc_gather_body,
        out_type=(
            jax.ShapeDtypeStruct((BF, K), jnp.float32),
            jax.ShapeDtypeStruct((BF,), jnp.float32),
        ),
        mesh=plsc.VectorSubcoreMesh(core_axis_name="c", subcore_axis_name="s"),
        scratch_types=[
            pltpu.VMEM((RPW,), jnp.int32),
            pltpu.VMEM((RPW, K), jnp.float32),
            pltpu.VMEM((RPW,), jnp.float32),
            pltpu.SemaphoreType.DMA,
        ],
        compiler_params=pltpu.CompilerParams(use_tc_tiling_on_sc=False),
    )


def _tc_dense_body(flat_ref, lin_ref, linb_ref, w1_ref, b1_ref, w2_ref,
                   b2_ref, w3_ref, b3_ref, out_ref):
    x = flat_ref[...]
    s = x[:, 0:K]
    for f in range(1, F):
        s = s + x[:, f * K:(f + 1) * K]
    sum_of_square = jnp.sum(s * s)
    square_of_sum = jnp.sum(x * x)
    interaction = 0.5 * (sum_of_square - square_of_sum)
    lin = lin_ref[...]
    line_out = jnp.sum(lin, axis=1, keepdims=True) + linb_ref[...]
    h = jnp.dot(x, w1_ref[...], preferred_element_type=jnp.float32)
    h = jnp.maximum(h + b1_ref[...], 0.0)
    h = jnp.dot(h, w2_ref[...], preferred_element_type=jnp.float32)
    h = jnp.maximum(h + b2_ref[...], 0.0)
    fnn = jnp.dot(h, w3_ref[...], preferred_element_type=jnp.float32)
    fnn = fnn + b3_ref[...]
    out_ref[...] = fnn + line_out + interaction


_tc_dense = pl.pallas_call(
    _tc_dense_body,
    out_shape=jax.ShapeDtypeStruct((B, 1), jnp.float32),
)


def kernel(inputs, emb_table, lin_table, lin_bias, W1, b1, W2, b2, W3, b3):
    flat_idx = (inputs + jnp.arange(F, dtype=jnp.int32)[None, :] * V)
    flat_idx = flat_idx.reshape(BF)
    emb_flat = emb_table.reshape(F * V, K)
    lin_flat = lin_table.reshape(F * V)
    emb_rows, lin_vals = _sc_gather()(flat_idx, emb_flat, lin_flat)
    flat = emb_rows.reshape(B, F * K)
    lin2 = lin_vals.reshape(B, F)
    return _tc_dense(flat, lin2, lin_bias, W1, b1, W2, b2, W3, b3)
